# Initial kernel scaffold; baseline (speedup 1.0000x reference)
import jax
import jax.numpy as jnp
from jax.experimental import pallas as pl


def kernel(queries, keys, values, Wq, bq, Wk, bk, Wv, bv, Wo, bo):
    raise NotImplementedError("write your pallas kernel here")


# speedup vs baseline: 4.0775x; 4.0775x over previous
import functools
import math

import jax
import jax.numpy as jnp
import numpy as np
from jax.experimental import pallas as pl
from jax.experimental.pallas import tpu as pltpu
from jax.experimental.pallas import tpu_sc as plsc

_B, _L, _D = 2, 4096, 768
_H, _HD = 12, 64
_N = 64
_C = _B * _D
_TOPK = max(int(math.log(_L)), 1)

_n = np.arange(_N)
_Fc = np.exp(-2j * np.pi * np.outer(_n, _n) / _N)
_TWc = np.exp(-2j * np.pi * np.outer(_n, _n) / _L)
_FR = np.ascontiguousarray(_Fc.real, np.float32)
_FI = np.ascontiguousarray(_Fc.imag, np.float32)
_TWR = np.ascontiguousarray(_TWc.real, np.float32)
_TWI = np.ascontiguousarray(_TWc.imag, np.float32)
_FRL = np.ascontiguousarray(_Fc.real / _L, np.float32)
_FIL = np.ascontiguousarray(_Fc.imag / _L, np.float32)

_LB = 512


def _proj_kernel(x_ref, wq_ref, wk_ref, wv_ref, bq_ref, bk_ref, bv_ref,
                 q_ref, k_ref, v_ref):
    dn = (((0,), (1,)), ((), ()))
    q_ref[...] = jax.lax.dot_general(wq_ref[...], x_ref[0, 0], dn,
                                     preferred_element_type=jnp.float32) + bq_ref[...]
    k_ref[...] = jax.lax.dot_general(wk_ref[...], x_ref[1, 0], dn,
                                     preferred_element_type=jnp.float32) + bk_ref[...]
    v_ref[...] = jax.lax.dot_general(wv_ref[...], x_ref[2, 0], dn,
                                     preferred_element_type=jnp.float32) + bv_ref[...]


def _project(qkv, Wq, Wk, Wv, bq, bk, bv):
    grid = (_B, _L // _LB)
    out_spec = pl.BlockSpec((_D, _LB), lambda b, i: (b, i))
    wspec = pl.BlockSpec((_D, _D), lambda b, i: (0, 0))
    bspec = pl.BlockSpec((_D, 1), lambda b, i: (0, 0))
    return pl.pallas_call(
        _proj_kernel,
        grid=grid,
        in_specs=[
            pl.BlockSpec((3, 1, _LB, _D), lambda b, i: (0, b, i, 0)),
            wspec, wspec, wspec, bspec, bspec, bspec,
        ],
        out_specs=[out_spec, out_spec, out_spec],
        out_shape=[jax.ShapeDtypeStruct((_C, _L), jnp.float32)] * 3,
    )(qkv, Wq, Wk, Wv, bq.reshape(_D, 1), bk.reshape(_D, 1), bv.reshape(_D, 1))


_CB = 64


def _corr_kernel(q_ref, k_ref, fr_ref, fi_ref, twr_ref, twi_ref,
                 frl_ref, fil_ref, w_ref, d_ref):
    fr, fi = fr_ref[...], fi_ref[...]
    twr, twi = twr_ref[...], twi_ref[...]

    def fwd(x):
        x3 = x.reshape(_CB, _N, _N)
        xt = jnp.transpose(x3, (0, 2, 1)).reshape(_CB * _N, _N)
        yr = jnp.dot(xt, fr, preferred_element_type=jnp.float32, precision=jax.lax.Precision.HIGHEST)
        yi = jnp.dot(xt, fi, preferred_element_type=jnp.float32, precision=jax.lax.Precision.HIGHEST)
        yr3 = yr.reshape(_CB, _N, _N)
        yi3 = yi.reshape(_CB, _N, _N)
        zr = yr3 * twr - yi3 * twi
        zi = yr3 * twi + yi3 * twr
        zr2 = jnp.transpose(zr, (0, 2, 1)).reshape(_CB * _N, _N)
        zi2 = jnp.transpose(zi, (0, 2, 1)).reshape(_CB * _N, _N)
        xr = (jnp.dot(zr2, fr, preferred_element_type=jnp.float32, precision=jax.lax.Precision.HIGHEST)
              - jnp.dot(zi2, fi, preferred_element_type=jnp.float32, precision=jax.lax.Precision.HIGHEST))
        xi = (jnp.dot(zr2, fi, preferred_element_type=jnp.float32, precision=jax.lax.Precision.HIGHEST)
              + jnp.dot(zi2, fr, preferred_element_type=jnp.float32, precision=jax.lax.Precision.HIGHEST))
        return xr, xi

    qr, qi = fwd(q_ref[...])
    kr, ki = fwd(k_ref[...])
    sr = qr * kr + qi * ki
    si = qi * kr - qr * ki
    tr = (jnp.dot(sr, fr, preferred_element_type=jnp.float32, precision=jax.lax.Precision.HIGHEST)
          + jnp.dot(si, fi, preferred_element_type=jnp.float32, precision=jax.lax.Precision.HIGHEST))
    ti = (jnp.dot(si, fr, preferred_element_type=jnp.float32, precision=jax.lax.Precision.HIGHEST)
          - jnp.dot(sr, fi, preferred_element_type=jnp.float32, precision=jax.lax.Precision.HIGHEST))
    tr3 = tr.reshape(_CB, _N, _N)
    ti3 = ti.reshape(_CB, _N, _N)
    ur = tr3 * twr + ti3 * twi
    ui = ti3 * twr - tr3 * twi
    ur2 = jnp.transpose(ur, (0, 2, 1)).reshape(_CB * _N, _N)
    ui2 = jnp.transpose(ui, (0, 2, 1)).reshape(_CB * _N, _N)
    vr = (jnp.dot(ur2, frl_ref[...], preferred_element_type=jnp.float32, precision=jax.lax.Precision.HIGHEST)
          + jnp.dot(ui2, fil_ref[...], preferred_element_type=jnp.float32, precision=jax.lax.Precision.HIGHEST))
    v3 = jnp.transpose(vr.reshape(_CB, _N, _N), (0, 2, 1))
    corr = v3.reshape(_CB, _L)

    iota2 = jax.lax.broadcasted_iota(jnp.int32, (_CB, _L), 1)
    work = corr
    vals, idxs = [], []
    for _ in range(_TOPK):
        m = jnp.max(work, axis=1, keepdims=True)
        amin = jnp.min(jnp.where(work == m, iota2, _L), axis=1,
                       keepdims=True)
        vals.append(m)
        idxs.append(amin)
        work = jnp.where(iota2 == amin, -jnp.inf, work)
    es = [jnp.exp(v - vals[0]) for v in vals]
    s = es[0]
    for e in es[1:]:
        s = s + e
    pad_w = jnp.zeros((_CB, 128 - _TOPK), jnp.float32)
    pad_d = jnp.zeros((_CB, 128 - _TOPK), jnp.int32)
    w_ref[...] = jnp.concatenate([e / s for e in es] + [pad_w], axis=1)
    d_ref[...] = jnp.concatenate(idxs + [pad_d], axis=1)


def _correlate_topk(qT, kT):
    grid = (_C // _CB,)
    fspec = pl.BlockSpec((_N, _N), lambda i: (0, 0))
    return pl.pallas_call(
        _corr_kernel,
        grid=grid,
        in_specs=[
            pl.BlockSpec((_CB, _L), lambda i: (i, 0)),
            pl.BlockSpec((_CB, _L), lambda i: (i, 0)),
            fspec, fspec, fspec, fspec, fspec, fspec,
        ],
        out_specs=[
            pl.BlockSpec((_CB, 128), lambda i: (i, 0)),
            pl.BlockSpec((_CB, 128), lambda i: (i, 0)),
        ],
        out_shape=[
            jax.ShapeDtypeStruct((_C, 128), jnp.float32),
            jax.ShapeDtypeStruct((_C, 128), jnp.int32),
        ],
    )(qT, kT, _FR, _FI, _TWR, _TWI, _FRL, _FIL)


def _combine(vT, w, d):
    info = plsc.get_sparse_core_info()
    nc, ns = info.num_cores, info.num_subcores
    nw = nc * ns
    nch = _C // nw
    mesh = plsc.VectorSubcoreMesh(core_axis_name="c", subcore_axis_name="s")

    @functools.partial(
        pl.kernel,
        out_type=jax.ShapeDtypeStruct((_C, _L), jnp.float32),
        mesh=mesh,
        scratch_types=[
            pltpu.VMEM((2 * _L,), jnp.float32),
            pltpu.VMEM((_L,), jnp.float32),
            pltpu.VMEM((16,), jnp.float32),
            pltpu.VMEM((16,), jnp.int32),
        ],
    )
    def sc_kernel(v_hbm, w_hbm, d_hbm, out_hbm, vv, out_v, w_v, d_v):
        wid = jax.lax.axis_index("s") * nc + jax.lax.axis_index("c")

        def chan_body(ci, _):
            c = wid * nch + ci
            pltpu.sync_copy(v_hbm.at[c], vv.at[pl.ds(0, _L)])
            pltpu.sync_copy(v_hbm.at[c], vv.at[pl.ds(_L, _L)])
            pltpu.sync_copy(w_hbm.at[c, pl.ds(0, 16)], w_v)
            pltpu.sync_copy(d_hbm.at[c, pl.ds(0, 16)], d_v)
            wvec = w_v[...]
            dvec = d_v[...]
            ws = [wvec[i] for i in range(_TOPK)]
            ds = [dvec[i] for i in range(_TOPK)]

            def out_body(j, _):
                base = j * 16
                acc = ws[0] * vv[pl.ds(ds[0] + base, 16)]
                for i in range(1, _TOPK):
                    acc = acc + ws[i] * vv[pl.ds(ds[i] + base, 16)]
                out_v[pl.ds(base, 16)] = acc
                return 0

            jax.lax.fori_loop(0, _L // 16, out_body, 0)
            pltpu.sync_copy(out_v, out_hbm.at[c])
            return 0

        jax.lax.fori_loop(0, nch, chan_body, 0)

    return sc_kernel(vT, w, d)


def _out_proj_kernel(x_ref, wo_ref, bo_ref, o_ref):
    dn = (((0,), (0,)), ((), ()))
    o_ref[0] = jax.lax.dot_general(x_ref[...], wo_ref[...], dn,
                                   preferred_element_type=jnp.float32) + bo_ref[...]


def _out_project(outT, Wo, bo):
    grid = (_B, _L // _LB)
    return pl.pallas_call(
        _out_proj_kernel,
        grid=grid,
        in_specs=[
            pl.BlockSpec((_D, _LB), lambda b, i: (b, i)),
            pl.BlockSpec((_D, _D), lambda b, i: (0, 0)),
            pl.BlockSpec((1, _D), lambda b, i: (0, 0)),
        ],
        out_specs=pl.BlockSpec((1, _LB, _D), lambda b, i: (b, i, 0)),
        out_shape=jax.ShapeDtypeStruct((_B, _L, _D), jnp.float32),
    )(outT, Wo, bo.reshape(1, _D))


def kernel(queries, keys, values, Wq, bq, Wk, bk, Wv, bv, Wo, bo):
    qkv = jnp.stack([queries, keys, values])
    qT, kT, vT = _project(qkv, Wq, Wk, Wv, bq, bk, bv)
    w, d = _correlate_topk(qT, kT)
    outT = _combine(vT, w, d)
    return _out_project(outT, Wo, bo)
